# Initial kernel scaffold; baseline (speedup 1.0000x reference)
import jax
import jax.numpy as jnp
from jax.experimental import pallas as pl


def kernel(input, indices):
    raise NotImplementedError("write your pallas kernel here")


# speedup vs baseline: 9.0962x; 9.0962x over previous
import functools

import jax
import jax.numpy as jnp
from jax import lax
from jax.experimental import pallas as pl
from jax.experimental.pallas import tpu as pltpu
from jax.experimental.pallas import tpu_sc as plsc

N, C, H_IN = 8, 768, 4096
H_OUT = 2 * H_IN
ROWS = N * C
NUM_WORKERS = 32
ROWS_PER_W = ROWS // NUM_WORKERS
LANES = 16
R_BLK = 8
SHIFT = 12
PMASK = (1 << SHIFT) - 1


NV = H_IN // 128


def _partner(w, M):
    Mv, Ml = M >> 7, M & 127
    x = w
    for b in (1, 2, 4, 8, 16):
        if Mv & b:
            v = x.reshape(R_BLK, NV // (2 * b), 2, b * 128)
            x = jnp.concatenate([v[:, :, 1:2, :], v[:, :, 0:1, :]], axis=2)
            x = x.reshape(R_BLK, H_IN)
    if Ml:
        x2 = x.reshape(R_BLK * NV, 128)
        lidx = lax.broadcasted_iota(jnp.int32, x2.shape, 1) ^ Ml
        x = jnp.take_along_axis(
            x2, lidx, axis=1, mode="promise_in_bounds"
        ).reshape(R_BLK, H_IN)
    return x


def _stage(w, M, L):
    P = _partner(w, M)
    sk = w >> SHIFT
    pk = P >> SHIFT
    il = lax.broadcasted_iota(jnp.int32, (R_BLK, H_IN), 1)
    lo = (il & L) == 0
    take = (lo & (pk < sk)) | (~lo & (sk < pk))
    return jnp.where(take, P, w)


def _sort_body(idx_ref, out_ref):
    w = idx_ref[...] * (1 << SHIFT) + lax.broadcasted_iota(
        jnp.int32, (R_BLK, H_IN), 1
    )
    size = 2
    while size <= H_IN:
        w = _stage(w, size - 1, size // 2)
        s = size // 4
        while s >= 1:
            w = _stage(w, s, s)
            s //= 2
        size *= 2
    out_ref[...] = w


_sort_tc = pl.pallas_call(
    _sort_body,
    grid=(ROWS // R_BLK,),
    in_specs=[pl.BlockSpec((R_BLK, H_IN), lambda i: (i, 0))],
    out_specs=pl.BlockSpec((R_BLK, H_IN), lambda i: (i, 0)),
    out_shape=jax.ShapeDtypeStruct((ROWS, H_IN), jnp.int32),
)


_mesh = plsc.VectorSubcoreMesh(core_axis_name="c", subcore_axis_name="s")


@functools.partial(
    pl.kernel,
    out_type=jax.ShapeDtypeStruct((ROWS, H_OUT), jnp.float32),
    mesh=_mesh,
    compiler_params=pltpu.CompilerParams(needs_layout_passes=False),
    scratch_types=[
        pltpu.VMEM((H_IN,), jnp.int32),
        pltpu.VMEM((H_IN,), jnp.float32),
        pltpu.VMEM((H_OUT,), jnp.float32),
    ],
)
def _scatter_sc(w_hbm, val_hbm, out_hbm, w_v, val_v, out_v):
    wid = lax.axis_index("s") * 2 + lax.axis_index("c")
    base = wid * ROWS_PER_W

    zero16 = jnp.zeros((LANES,), jnp.float32)

    def row_body(r, carry):
        row = base + r
        pltpu.sync_copy(w_hbm.at[row], w_v)
        pltpu.sync_copy(val_hbm.at[row], val_v)

        def zero_body(i, c):
            out_v[pl.ds(i * LANES, LANES)] = zero16
            return c

        lax.fori_loop(0, H_OUT // LANES, zero_body, 0, unroll=8)

        def scat_body(j, c):
            wvec = w_v[pl.ds(j * LANES, LANES)]
            k = wvec >> SHIFT
            h = wvec & PMASK
            dvec = plsc.load_gather(val_v, [h])
            plsc.store_scatter(out_v, [k], dvec)
            return c

        lax.fori_loop(0, H_IN // LANES, scat_body, 0, unroll=8)

        pltpu.sync_copy(out_v, out_hbm.at[row])
        return carry

    lax.fori_loop(0, ROWS_PER_W, row_body, 0)


def kernel(input, indices):
    idx = indices.astype(jnp.int32).reshape(ROWS, H_IN)
    val = input.reshape(ROWS, H_IN)
    w = _sort_tc(idx)
    out = _scatter_sc(w, val)
    return out.reshape(N, C, H_OUT)


# speedup vs baseline: 10.3238x; 1.1350x over previous
import functools

import jax
import jax.numpy as jnp
from jax import lax
from jax.experimental import pallas as pl
from jax.experimental.pallas import tpu as pltpu
from jax.experimental.pallas import tpu_sc as plsc

N, C, H_IN = 8, 768, 4096
H_OUT = 2 * H_IN
ROWS = N * C
NUM_WORKERS = 32
ROWS_PER_W = ROWS // NUM_WORKERS
LANES = 16
R_BLK = 8
SHIFT = 12
PMASK = (1 << SHIFT) - 1


def _sort_body(idx_ref, out_ref):
    il = lax.broadcasted_iota(jnp.int32, (R_BLK, H_IN), 1)
    w = idx_ref[...] * (1 << SHIFT) + il

    masks = {}

    def lomask(L):
        if L not in masks:
            masks[L] = (il & L) == 0
        return masks[L]

    def stage_xor(w, s):
        m = lomask(s)
        a = pltpu.roll(w, H_IN - s, 1)
        b = pltpu.roll(w, s, 1)
        k = w >> SHIFT
        t_lo = jnp.where((a >> SHIFT) < k, a, w)
        t_hi = jnp.where(k < (b >> SHIFT), b, w)
        return jnp.where(m, t_lo, t_hi)

    def stage_rev(w, size):
        p = w
        b = 1
        while b < size:
            p = jnp.where(
                lomask(b), pltpu.roll(p, H_IN - b, 1), pltpu.roll(p, b, 1)
            )
            b *= 2
        m = lomask(size // 2)
        k = w >> SHIFT
        pk = p >> SHIFT
        t_lo = jnp.where(pk < k, p, w)
        t_hi = jnp.where(k < pk, p, w)
        return jnp.where(m, t_lo, t_hi)

    size = 2
    while size <= H_IN:
        w = stage_rev(w, size)
        s = size // 4
        while s >= 1:
            w = stage_xor(w, s)
            s //= 2
        size *= 2
    out_ref[...] = w


_sort_tc = pl.pallas_call(
    _sort_body,
    grid=(ROWS // R_BLK,),
    in_specs=[pl.BlockSpec((R_BLK, H_IN), lambda i: (i, 0))],
    out_specs=pl.BlockSpec((R_BLK, H_IN), lambda i: (i, 0)),
    out_shape=jax.ShapeDtypeStruct((ROWS, H_IN), jnp.int32),
)


_mesh = plsc.VectorSubcoreMesh(core_axis_name="c", subcore_axis_name="s")


@functools.partial(
    pl.kernel,
    out_type=jax.ShapeDtypeStruct((ROWS, H_OUT), jnp.float32),
    mesh=_mesh,
    compiler_params=pltpu.CompilerParams(needs_layout_passes=False),
    scratch_types=[
        pltpu.VMEM((H_IN,), jnp.int32),
        pltpu.VMEM((H_IN,), jnp.float32),
        pltpu.VMEM((H_OUT,), jnp.float32),
    ],
)
def _scatter_sc(w_hbm, val_hbm, out_hbm, w_v, val_v, out_v):
    wid = lax.axis_index("s") * 2 + lax.axis_index("c")
    base = wid * ROWS_PER_W

    zero16 = jnp.zeros((LANES,), jnp.float32)

    def row_body(r, carry):
        row = base + r
        pltpu.sync_copy(w_hbm.at[row], w_v)
        pltpu.sync_copy(val_hbm.at[row], val_v)

        def zero_body(i, c):
            out_v[pl.ds(i * LANES, LANES)] = zero16
            return c

        lax.fori_loop(0, H_OUT // LANES, zero_body, 0, unroll=8)

        def scat_body(j, c):
            wvec = w_v[pl.ds(j * LANES, LANES)]
            k = wvec >> SHIFT
            h = wvec & PMASK
            dvec = plsc.load_gather(val_v, [h])
            plsc.store_scatter(out_v, [k], dvec)
            return c

        lax.fori_loop(0, H_IN // LANES, scat_body, 0, unroll=8)

        pltpu.sync_copy(out_v, out_hbm.at[row])
        return carry

    lax.fori_loop(0, ROWS_PER_W, row_body, 0)


def kernel(input, indices):
    idx = indices.astype(jnp.int32).reshape(ROWS, H_IN)
    val = input.reshape(ROWS, H_IN)
    w = _sort_tc(idx)
    out = _scatter_sc(w, val)
    return out.reshape(N, C, H_OUT)
